# baseline (speedup 1.0000x reference)
import jax
import jax.numpy as jnp
from jax.experimental import pallas as pl


def _bcast_copy_kernel(w_ref, o_ref):
    o_ref[...] = w_ref[...][None]


def kernel(x, weight):
    B, S = x.shape
    M, D = weight.shape
    BS = 2048
    return pl.pallas_call(
        _bcast_copy_kernel,
        grid=(S // BS, B),
        in_specs=[pl.BlockSpec((BS, D), lambda s, b: (s, 0))],
        out_specs=pl.BlockSpec((1, BS, D), lambda s, b: (b, s, 0)),
        out_shape=jax.ShapeDtypeStruct((B, S, D), weight.dtype),
    )(weight)
